# baseline (speedup 1.0000x reference)
import functools

import jax
import jax.numpy as jnp
from jax import lax
from jax.experimental import pallas as pl
from jax.experimental.pallas import tpu as pltpu
from jax.experimental.pallas import tpu_sc as plsc

_N = 100000
_D = 128
_G = 128
_B = 256
_S = 128
_NSUB = _N // _S
_TAIL = _N - _NSUB * _S
_NW = 32
_MAXJ = -(-_NSUB // _NW)

_RB = 2000
_NCB = _N // _RB


def _sc_body(x_hbm, batch_hbm, acc_out,
             xbuf, idbuf, xtail, idtail, zrow, acc_sh):
    cid = lax.axis_index("c")
    sid = lax.axis_index("s")
    wid = sid * 2 + cid

    zero16 = jnp.zeros((16,), jnp.float32)

    def _init_z(i, carry):
        for g in range(_D // 16):
            zrow[i, pl.ds(g * 16, 16)] = zero16
        return carry

    lax.fori_loop(0, 16, _init_z, 0)

    pltpu.sync_copy(zrow, acc_sh.at[pl.ds(sid * 16, 16), :])
    plsc.subcore_barrier()

    def _chunk(j, carry):
        k = wid + _NW * j

        @pl.when(k < _NSUB)
        def _():
            pltpu.sync_copy(batch_hbm.at[pl.ds(k * _S, _S)], idbuf)
            pltpu.sync_copy(x_hbm.at[pl.ds(k * _S, _S), :], xbuf)
            pltpu.sync_copy(xbuf, acc_sh.at[idbuf], add=True)

        return carry

    lax.fori_loop(0, _MAXJ, _chunk, 0)

    @pl.when(wid == _NW - 1)
    def _tail():
        pltpu.sync_copy(batch_hbm.at[pl.ds(_NSUB * _S, _TAIL)], idtail)
        pltpu.sync_copy(x_hbm.at[pl.ds(_NSUB * _S, _TAIL), :], xtail)
        pltpu.sync_copy(xtail, acc_sh.at[idtail], add=True)

    plsc.subcore_barrier()
    pltpu.sync_copy(acc_sh.at[pl.ds(sid * 16, 16), :],
                    acc_out.at[cid, pl.ds(sid * 16, 16), :])


_sc_segsum = functools.partial(
    pl.kernel,
    mesh=plsc.VectorSubcoreMesh(core_axis_name="c", subcore_axis_name="s"),
    out_type=jax.ShapeDtypeStruct((2, _B, _D), jnp.float32),
    scratch_types=[
        pltpu.VMEM((_S, _D), jnp.float32),
        pltpu.VMEM((_S,), jnp.int32),
        pltpu.VMEM((_TAIL, _D), jnp.float32),
        pltpu.VMEM((_TAIL,), jnp.int32),
        pltpu.VMEM((16, _D), jnp.float32),
        pltpu.VMEM_SHARED((_B, _D), jnp.float32),
    ],
)(_sc_body)


def _mlp_body(batch_ref, acc_ref, u_ref, W1_ref, b1_ref, W2_ref, b2_ref,
              out_ref, cnt):
    i = pl.program_id(0)

    @pl.when(i == 0)
    def _init():
        cnt[...] = jnp.zeros_like(cnt)

    @pl.when(i < _NCB)
    def _count():
        ids = batch_ref[0, 0, :]
        onehot = (jax.lax.broadcasted_iota(jnp.int32, (_B, _RB), 0)
                  == ids[None, :]).astype(jnp.float32)
        cnt[...] += jnp.sum(onehot, axis=1, keepdims=True)

    @pl.when(i == _NCB)
    def _finish():
        sums = acc_ref[0] + acc_ref[1]
        pooled = sums / jnp.maximum(cnt[...], 1.0)
        h = jnp.maximum(
            jnp.dot(u_ref[...], W1_ref[0:_G, :],
                    preferred_element_type=jnp.float32)
            + jnp.dot(pooled, W1_ref[_G:_G + _D, :],
                      preferred_element_type=jnp.float32)
            + b1_ref[...], 0.0)
        out_ref[...] = (jnp.dot(h, W2_ref[...],
                                preferred_element_type=jnp.float32)
                        + b2_ref[...])


def kernel(x, edge_index, edge_attr, u, batch, W1, b1, W2, b2):
    del edge_index, edge_attr
    batch_i32 = batch.astype(jnp.int32)
    acc2 = _sc_segsum(x, batch_i32)
    batch3d = batch_i32.reshape(_NCB, 1, _RB)
    return pl.pallas_call(
        _mlp_body,
        grid=(_NCB + 1,),
        in_specs=[
            pl.BlockSpec((1, 1, _RB), lambda i: (jnp.minimum(i, _NCB - 1), 0, 0)),
            pl.BlockSpec((2, _B, _D), lambda i: (0, 0, 0)),
            pl.BlockSpec((_B, _G), lambda i: (0, 0)),
            pl.BlockSpec((_G + _D, _G), lambda i: (0, 0)),
            pl.BlockSpec((1, _G), lambda i: (0, 0)),
            pl.BlockSpec((_G, _G), lambda i: (0, 0)),
            pl.BlockSpec((1, _G), lambda i: (0, 0)),
        ],
        out_specs=pl.BlockSpec((_B, _G), lambda i: (0, 0)),
        out_shape=jax.ShapeDtypeStruct((_B, _G), jnp.float32),
        scratch_shapes=[pltpu.VMEM((_B, 1), jnp.float32)],
    )(batch3d, acc2, u, W1, b1.reshape(1, _G), W2, b2.reshape(1, _G))


# speedup vs baseline: 2.0680x; 2.0680x over previous
import functools

import jax
import jax.numpy as jnp
from jax import lax
from jax.experimental import pallas as pl
from jax.experimental.pallas import tpu as pltpu
from jax.experimental.pallas import tpu_sc as plsc

_N = 100000
_D = 128
_G = 128
_B = 256
_S = 128
_NSUB = _N // _S
_TAIL = _N - _NSUB * _S
_NW = 32
_MAXJ = -(-_NSUB // _NW)
_HALF = _MAXJ // 2

_RB = 2000
_NCB = _N // _RB


def _sc_body(x_hbm, batch_hbm, acc_out,
             xbufa, xbufb, idbufa, idbufb, xtail, idtail, zrow, acc_sh,
             sema, semb):
    cid = lax.axis_index("c")
    sid = lax.axis_index("s")
    wid = sid * 2 + cid

    zero16 = jnp.zeros((16,), jnp.float32)

    def _init_z(i, carry):
        for g in range(_D // 16):
            zrow[i, pl.ds(g * 16, 16)] = zero16
        return carry

    lax.fori_loop(0, 16, _init_z, 0)

    pltpu.sync_copy(zrow, acc_sh.at[pl.ds(sid * 16, 16), :])
    plsc.subcore_barrier()

    def _start(k, idbuf, xbuf, sem):
        @pl.when(k < _NSUB)
        def _():
            pltpu.async_copy(batch_hbm.at[pl.ds(k * _S, _S)], idbuf, sem)
            pltpu.async_copy(x_hbm.at[pl.ds(k * _S, _S), :], xbuf, sem)

    def _finish(k, idbuf, xbuf, sem):
        @pl.when(k < _NSUB)
        def _():
            pltpu.make_async_copy(batch_hbm.at[pl.ds(k * _S, _S)],
                                  idbuf, sem).wait()
            pltpu.make_async_copy(x_hbm.at[pl.ds(k * _S, _S), :],
                                  xbuf, sem).wait()
            pltpu.sync_copy(xbuf, acc_sh.at[idbuf], add=True)

    _start(wid, idbufa, xbufa, sema)

    def _round(jj, carry):
        ka = wid + _NW * (2 * jj)
        kb = wid + _NW * (2 * jj + 1)
        ka2 = wid + _NW * (2 * jj + 2)
        _start(kb, idbufb, xbufb, semb)
        _finish(ka, idbufa, xbufa, sema)
        _start(ka2, idbufa, xbufa, sema)
        _finish(kb, idbufb, xbufb, semb)
        return carry

    lax.fori_loop(0, _HALF, _round, 0)
    _finish(wid + _NW * (2 * _HALF), idbufa, xbufa, sema)

    @pl.when(wid == _NW - 1)
    def _tail():
        pltpu.sync_copy(batch_hbm.at[pl.ds(_NSUB * _S, _TAIL)], idtail)
        pltpu.sync_copy(x_hbm.at[pl.ds(_NSUB * _S, _TAIL), :], xtail)
        pltpu.sync_copy(xtail, acc_sh.at[idtail], add=True)

    plsc.subcore_barrier()
    pltpu.sync_copy(acc_sh.at[pl.ds(sid * 16, 16), :],
                    acc_out.at[cid, pl.ds(sid * 16, 16), :])


_sc_segsum = functools.partial(
    pl.kernel,
    mesh=plsc.VectorSubcoreMesh(core_axis_name="c", subcore_axis_name="s"),
    out_type=jax.ShapeDtypeStruct((2, _B, _D), jnp.float32),
    scratch_types=[
        pltpu.VMEM((_S, _D), jnp.float32),
        pltpu.VMEM((_S, _D), jnp.float32),
        pltpu.VMEM((_S,), jnp.int32),
        pltpu.VMEM((_S,), jnp.int32),
        pltpu.VMEM((_TAIL, _D), jnp.float32),
        pltpu.VMEM((_TAIL,), jnp.int32),
        pltpu.VMEM((16, _D), jnp.float32),
        pltpu.VMEM_SHARED((_B, _D), jnp.float32),
        pltpu.SemaphoreType.DMA,
        pltpu.SemaphoreType.DMA,
    ],
)(_sc_body)


def _count_body(batch_ref, cnt_ref):
    i = pl.program_id(0)

    @pl.when(i == 0)
    def _init():
        cnt_ref[...] = jnp.zeros_like(cnt_ref)

    ids = batch_ref[0, 0, :]
    onehot = (jax.lax.broadcasted_iota(jnp.int32, (_B, _RB), 0)
              == ids[None, :]).astype(jnp.float32)
    cnt_ref[...] += jnp.sum(onehot, axis=1, keepdims=True)


def _mlp_body(acc_ref, cnt_ref, u_ref, W1_ref, b1_ref, W2_ref, b2_ref,
              out_ref):
    sums = acc_ref[0] + acc_ref[1]
    pooled = sums / jnp.maximum(cnt_ref[...], 1.0)
    h = jnp.maximum(
        jnp.dot(u_ref[...], W1_ref[0:_G, :],
                preferred_element_type=jnp.float32)
        + jnp.dot(pooled, W1_ref[_G:_G + _D, :],
                  preferred_element_type=jnp.float32)
        + b1_ref[...], 0.0)
    out_ref[...] = (jnp.dot(h, W2_ref[...],
                            preferred_element_type=jnp.float32)
                    + b2_ref[...])


def kernel(x, edge_index, edge_attr, u, batch, W1, b1, W2, b2):
    del edge_index, edge_attr
    batch_i32 = batch.astype(jnp.int32)
    acc2 = _sc_segsum(x, batch_i32)

    batch3d = batch_i32.reshape(_NCB, 1, _RB)
    cnt = pl.pallas_call(
        _count_body,
        grid=(_NCB,),
        in_specs=[pl.BlockSpec((1, 1, _RB), lambda i: (i, 0, 0))],
        out_specs=pl.BlockSpec((_B, 1), lambda i: (0, 0)),
        out_shape=jax.ShapeDtypeStruct((_B, 1), jnp.float32),
    )(batch3d)

    return pl.pallas_call(
        _mlp_body,
        out_shape=jax.ShapeDtypeStruct((_B, _G), jnp.float32),
    )(acc2, cnt, u, W1, b1.reshape(1, _G), W2, b2.reshape(1, _G))
